# Initial kernel scaffold; baseline (speedup 1.0000x reference)
import jax
import jax.numpy as jnp
from jax.experimental import pallas as pl


def kernel(x, edge_index, enc_lin_W, enc_lin_b, enc_conv1_W, enc_conv1_b, enc_conv2_W, enc_conv2_b, enc_conv3_W, enc_conv3_b, enc_post_W, enc_post_b, dec_conv1_W, dec_conv1_b, dec_conv2_W, dec_conv2_b, dec_conv3_W, dec_conv3_b, dec_post_W, dec_post_b):
    raise NotImplementedError("write your pallas kernel here")


# speedup vs baseline: 8.6428x; 8.6428x over previous
import jax
import jax.numpy as jnp
from jax.experimental import pallas as pl
from jax.experimental.pallas import tpu as pltpu
from jax.experimental.pallas import tpu_sc as plsc

_N = 50000
_E = 800000
_EP_ROWS = 6400
_EP = _EP_ROWS * 128
_ACC_ROWS = 50048
_SUB_ROWS = 3128
_LAST_ROWS = 3080
_ZB_ROWS = 391

_AGG_SUB_CHUNKS = 400
_AGG_NB = 4
_AGG_ITERS = 100

_DEG_SUB_CHUNKS = 200
_DEG_NB = 8
_DEG_ITERS = 25

_R = 2000
_G = 25

_HIGH = jax.lax.Precision.HIGHEST
_sc_params = pltpu.CompilerParams(use_tc_tiling_on_sc=False)
_mesh = plsc.VectorSubcoreMesh(core_axis_name="c", subcore_axis_name="s")


def _zero_acc(acc, zbuf, s, width):
    zeros16 = jnp.zeros((16,), jnp.float32)

    @pl.loop(0, _ZB_ROWS)
    def _(i):
        for w0 in range(0, width, 16):
            zbuf[i, pl.ds(w0, 16)] = zeros16

    r0 = s * _SUB_ROWS
    for q in range(8):
        pltpu.sync_copy(zbuf, acc.at[pl.ds(r0 + q * _ZB_ROWS, _ZB_ROWS)])


def _writeback(acc, out, s):
    r0 = s * _SUB_ROWS

    @pl.when(s != 15)
    def _():
        pltpu.sync_copy(acc.at[pl.ds(r0, _SUB_ROWS)],
                        out.at[pl.ds(r0, _SUB_ROWS)])

    @pl.when(s == 15)
    def _():
        pltpu.sync_copy(acc.at[pl.ds(15 * _SUB_ROWS, _LAST_ROWS)],
                        out.at[pl.ds(15 * _SUB_ROWS, _LAST_ROWS)])


def _agg_body(hs_a, hs_b, src_r, dst_r, s_a, s_b, acc, zbuf, sidx, didx, gath):
    c = jax.lax.axis_index("c")
    s = jax.lax.axis_index("s")
    _zero_acc(acc, zbuf, s, 32)
    plsc.subcore_barrier()

    def edge_pass(hs):
        @pl.loop(0, _AGG_ITERS)
        def _(it):
            row0 = s * _AGG_SUB_CHUNKS + it * _AGG_NB
            pltpu.sync_copy(src_r.at[pl.ds(row0, _AGG_NB)], sidx)
            pltpu.sync_copy(dst_r.at[pl.ds(row0, _AGG_NB)], didx)
            for j in range(_AGG_NB):
                pltpu.sync_copy(hs.at[sidx.at[j]],
                                gath.at[pl.ds(j * 128, 128)])
            for j in range(_AGG_NB):
                pltpu.sync_copy(gath.at[pl.ds(j * 128, 128)],
                                acc.at[didx.at[j]], add=True)

    @pl.when(c == 0)
    def _():
        edge_pass(hs_a)

    @pl.when(c == 1)
    def _():
        edge_pass(hs_b)

    plsc.subcore_barrier()

    @pl.when(c == 0)
    def _():
        _writeback(acc, s_a, s)

    @pl.when(c == 1)
    def _():
        _writeback(acc, s_b, s)


@jax.jit
def _agg(hs_a, hs_b, src_r, dst_r):
    f32 = jnp.float32
    return pl.kernel(
        _agg_body,
        out_type=(jax.ShapeDtypeStruct((_N, 32), f32),
                  jax.ShapeDtypeStruct((_N, 32), f32)),
        mesh=_mesh,
        compiler_params=_sc_params,
        scratch_types=[
            pltpu.VMEM_SHARED((_ACC_ROWS, 32), f32),
            pltpu.VMEM((_ZB_ROWS, 32), f32),
            pltpu.VMEM((_AGG_NB, 128), jnp.int32),
            pltpu.VMEM((_AGG_NB, 128), jnp.int32),
            pltpu.VMEM((_AGG_NB * 128, 32), f32),
        ],
    )(hs_a, hs_b, src_r, dst_r)


def _deg_body(dst_r, degp_a, degp_b, acc, zbuf, didx, ones_buf):
    c = jax.lax.axis_index("c")
    s = jax.lax.axis_index("s")
    ones16 = jnp.ones((16,), jnp.float32)

    @pl.loop(0, 64)
    def _(i):
        ones_buf[i, pl.ds(0, 16)] = ones16

    _zero_acc(acc, zbuf, s, 8)
    plsc.subcore_barrier()

    @pl.loop(0, _DEG_ITERS)
    def _(it):
        row0 = (c * 16 + s) * _DEG_SUB_CHUNKS + it * _DEG_NB
        pltpu.sync_copy(dst_r.at[pl.ds(row0, _DEG_NB)], didx)
        for j in range(_DEG_NB):
            pltpu.sync_copy(ones_buf, acc.at[didx.at[j]], add=True)

    plsc.subcore_barrier()

    @pl.when(c == 0)
    def _():
        _writeback(acc, degp_a, s)

    @pl.when(c == 1)
    def _():
        _writeback(acc, degp_b, s)


@jax.jit
def _deg(dst_r):
    f32 = jnp.float32
    return pl.kernel(
        _deg_body,
        out_type=(jax.ShapeDtypeStruct((_N, 8), f32),
                  jax.ShapeDtypeStruct((_N, 8), f32)),
        mesh=_mesh,
        compiler_params=_sc_params,
        scratch_types=[
            pltpu.VMEM_SHARED((_ACC_ROWS, 8), f32),
            pltpu.VMEM((_ZB_ROWS, 8), f32),
            pltpu.VMEM((_DEG_NB, 128), jnp.int32),
            pltpu.VMEM((128, 8), f32),
        ],
    )(dst_r)


def _row_spec(width):
    return pl.BlockSpec((_R, width), lambda i: (i, 0))


def _full_spec(shape):
    return pl.BlockSpec(shape, lambda i: tuple(0 for _ in shape))


def _k0_body(x, dpa, dpb, wlin, blin, w1, hs_a, hs_b, degq):
    deg = dpa[...] + dpb[...] + 1.0
    degq[...] = deg
    dis = jax.lax.rsqrt(deg[:, :1])
    t0 = jnp.tanh(jnp.dot(x[...], wlin[...], precision=_HIGH) + blin[0])
    hs = jnp.dot(t0, w1[...], precision=_HIGH) * dis
    hs_a[...] = hs[:, :32]
    hs_b[...] = hs[:, 32:]


@jax.jit
def _k0(x, dpa, dpb, wlin, blin, w1):
    f32 = jnp.float32
    return pl.pallas_call(
        _k0_body,
        grid=(_G,),
        in_specs=[_row_spec(128), _row_spec(8), _row_spec(8),
                  _full_spec((128, 64)), _full_spec((1, 64)),
                  _full_spec((64, 64))],
        out_specs=(_row_spec(32), _row_spec(32), _row_spec(8)),
        out_shape=(jax.ShapeDtypeStruct((_N, 32), f32),
                   jax.ShapeDtypeStruct((_N, 32), f32),
                   jax.ShapeDtypeStruct((_N, 8), f32)),
    )(x, dpa, dpb, wlin, blin, w1)


def _act(sa, sb, ha, hb, degq, b):
    dis = jax.lax.rsqrt(degq[...][:, :1])
    full = jnp.concatenate([sa[...] + ha[...], sb[...] + hb[...]], axis=1)
    return jnp.tanh(full * dis + b[0]), dis


def _kmid_body(sa, sb, ha, hb, degq, b, w, oa, ob):
    t, dis = _act(sa, sb, ha, hb, degq, b)
    hs = jnp.dot(t, w[...], precision=_HIGH) * dis
    oa[...] = hs[:, :32]
    ob[...] = hs[:, 32:]


@jax.jit
def _kmid(sa, sb, ha, hb, degq, b, w):
    f32 = jnp.float32
    return pl.pallas_call(
        _kmid_body,
        grid=(_G,),
        in_specs=[_row_spec(32), _row_spec(32), _row_spec(32), _row_spec(32),
                  _row_spec(8), _full_spec((1, 64)), _full_spec((64, 64))],
        out_specs=(_row_spec(32), _row_spec(32)),
        out_shape=(jax.ShapeDtypeStruct((_N, 32), f32),
                   jax.ShapeDtypeStruct((_N, 32), f32)),
    )(sa, sb, ha, hb, degq, b, w)


def _k3_body(sa, sb, ha, hb, degq, b3, wpost, bpost, wd1,
             h_l, z, oa, ob):
    t, dis = _act(sa, sb, ha, hb, degq, b3)
    h_l[...] = t
    zz = jnp.dot(t, wpost[...], precision=_HIGH) + bpost[0]
    z[...] = zz
    hs = jnp.dot(zz, wd1[...], precision=_HIGH) * dis
    oa[...] = hs[:, :32]
    ob[...] = hs[:, 32:]


@jax.jit
def _k3(sa, sb, ha, hb, degq, b3, wpost, bpost, wd1):
    f32 = jnp.float32
    return pl.pallas_call(
        _k3_body,
        grid=(_G,),
        in_specs=[_row_spec(32), _row_spec(32), _row_spec(32), _row_spec(32),
                  _row_spec(8), _full_spec((1, 64)), _full_spec((64, 64)),
                  _full_spec((1, 64)), _full_spec((64, 64))],
        out_specs=(_row_spec(64), _row_spec(64), _row_spec(32), _row_spec(32)),
        out_shape=(jax.ShapeDtypeStruct((_N, 64), f32),
                   jax.ShapeDtypeStruct((_N, 64), f32),
                   jax.ShapeDtypeStruct((_N, 32), f32),
                   jax.ShapeDtypeStruct((_N, 32), f32)),
    )(sa, sb, ha, hb, degq, b3, wpost, bpost, wd1)


def _k5_body(sa, sb, ha, hb, degq, b, oa, ob):
    t, dis = _act(sa, sb, ha, hb, degq, b)
    hs = t * dis
    oa[...] = hs[:, :32]
    ob[...] = hs[:, 32:]


@jax.jit
def _k5(sa, sb, ha, hb, degq, b):
    f32 = jnp.float32
    return pl.pallas_call(
        _k5_body,
        grid=(_G,),
        in_specs=[_row_spec(32), _row_spec(32), _row_spec(32), _row_spec(32),
                  _row_spec(8), _full_spec((1, 64))],
        out_specs=(_row_spec(32), _row_spec(32)),
        out_shape=(jax.ShapeDtypeStruct((_N, 32), f32),
                   jax.ShapeDtypeStruct((_N, 32), f32)),
    )(sa, sb, ha, hb, degq, b)


def _k6_body(sa, sb, ha, hb, degq, w3d, b3d, wp, bp, hp, recon):
    dis = jax.lax.rsqrt(degq[...][:, :1])
    agg = jnp.concatenate([sa[...] + ha[...], sb[...] + hb[...]], axis=1) * dis
    h = jnp.dot(agg, w3d[...], precision=_HIGH) + b3d[0]
    hp[...] = h
    recon[...] = jnp.dot(h, wp[...], precision=_HIGH) + bp[0]


@jax.jit
def _k6(sa, sb, ha, hb, degq, w3d, b3d, wp, bp):
    f32 = jnp.float32
    return pl.pallas_call(
        _k6_body,
        grid=(_G,),
        in_specs=[_row_spec(32), _row_spec(32), _row_spec(32), _row_spec(32),
                  _row_spec(8), _full_spec((64, 128)), _full_spec((1, 128)),
                  _full_spec((128, 128)), _full_spec((1, 128))],
        out_specs=(_row_spec(128), _row_spec(128)),
        out_shape=(jax.ShapeDtypeStruct((_N, 128), f32),
                   jax.ShapeDtypeStruct((_N, 128), f32)),
    )(sa, sb, ha, hb, degq, w3d, b3d, wp, bp)


def kernel(x, edge_index,
           enc_lin_W, enc_lin_b,
           enc_conv1_W, enc_conv1_b, enc_conv2_W, enc_conv2_b,
           enc_conv3_W, enc_conv3_b,
           enc_post_W, enc_post_b,
           dec_conv1_W, dec_conv1_b, dec_conv2_W, dec_conv2_b,
           dec_conv3_W, dec_conv3_b,
           dec_post_W, dec_post_b):
    pad = _EP - _E
    src_r = jnp.concatenate(
        [edge_index[0], jnp.zeros((pad,), jnp.int32)]).reshape(_EP_ROWS, 128)
    dst_r = jnp.concatenate(
        [edge_index[1], jnp.full((pad,), _N, jnp.int32)]).reshape(_EP_ROWS, 128)

    row = lambda v: v.reshape(1, -1)
    dpa, dpb = _deg(dst_r)
    h1a, h1b, degq = _k0(x, dpa, dpb, enc_lin_W, row(enc_lin_b), enc_conv1_W)
    s1a, s1b = _agg(h1a, h1b, src_r, dst_r)
    h2a, h2b = _kmid(s1a, s1b, h1a, h1b, degq, row(enc_conv1_b), enc_conv2_W)
    s2a, s2b = _agg(h2a, h2b, src_r, dst_r)
    h3a, h3b = _kmid(s2a, s2b, h2a, h2b, degq, row(enc_conv2_b), enc_conv3_W)
    s3a, s3b = _agg(h3a, h3b, src_r, dst_r)
    h_l, z, h4a, h4b = _k3(s3a, s3b, h3a, h3b, degq, row(enc_conv3_b),
                           enc_post_W, row(enc_post_b), dec_conv1_W)
    s4a, s4b = _agg(h4a, h4b, src_r, dst_r)
    h5a, h5b = _kmid(s4a, s4b, h4a, h4b, degq, row(dec_conv1_b), dec_conv2_W)
    s5a, s5b = _agg(h5a, h5b, src_r, dst_r)
    h6a, h6b = _k5(s5a, s5b, h5a, h5b, degq, row(dec_conv2_b))
    s6a, s6b = _agg(h6a, h6b, src_r, dst_r)
    h_prime_l, recon = _k6(s6a, s6b, h6a, h6b, degq, dec_conv3_W,
                           row(dec_conv3_b), dec_post_W, row(dec_post_b))
    return (recon, z, h_l, h_prime_l)
